# Initial kernel scaffold; baseline (speedup 1.0000x reference)
import jax
import jax.numpy as jnp
from jax.experimental import pallas as pl


def kernel(features, neighbors, first_pcd_length, second_pcd_length):
    raise NotImplementedError("write your pallas kernel here")


# speedup vs baseline: 1.5304x; 1.5304x over previous
import functools

import jax
import jax.numpy as jnp
from jax import lax
from jax.experimental import pallas as pl
from jax.experimental.pallas import tpu as pltpu
from jax.experimental.pallas import tpu_sc as plsc

N = 10000
K = 32
D = 128

_INFO = plsc.get_sparse_core_info()
NC = _INFO.num_cores
NS = _INFO.num_subcores
NW = NC * NS
P = 320
NPAD = P * NW
NBUF = 4
LANES = 16


def _stats_body(f_ref, sum_ref, max_ref):
    f = f_ref[:]
    sum_ref[:] = jnp.sum(f, axis=1, keepdims=True)
    max_ref[:] = jnp.max(f).reshape(1, 1)


def _row_stats(features):
    return pl.pallas_call(
        _stats_body,
        out_shape=(
            jax.ShapeDtypeStruct((N, 1), jnp.float32),
            jax.ShapeDtypeStruct((1, 1), jnp.float32),
        ),
    )(features)


def _sc_body(feat_hbm, nbrs_hbm, rowsum_hbm,
             acc_hbm, cnt_hbm,
             nbrs_v, rowsum_v, rows_v, acc_v, cnt_v, *sems):
    wid = lax.axis_index("s") * NC + lax.axis_index("c")
    base = wid * P

    pltpu.sync_copy(nbrs_hbm.at[pl.ds(base * K, P * K)], nbrs_v)
    pltpu.sync_copy(rowsum_hbm, rowsum_v)

    lanes = lax.iota(jnp.int32, LANES)

    def count_group(g, _):
        flatbase = (g * LANES + lanes) * K
        cnt = jnp.zeros((LANES,), jnp.float32)
        for k in range(K):
            ids = plsc.load_gather(nbrs_v, [flatbase + k])
            vals = plsc.load_gather(rowsum_v, [ids])
            cnt = cnt + jnp.where(vals != 0.0, 1.0, 0.0)
        cnt_v[pl.ds(g * LANES, LANES)] = cnt
        return _

    lax.fori_loop(0, P // LANES, count_group, None)

    def idx_at(i):
        return nbrs_v.at[pl.ds(i * K, K)]

    for b in range(NBUF):
        pltpu.async_copy(feat_hbm.at[idx_at(b)], rows_v.at[b], sems[b])

    waiters = [
        pltpu.make_async_copy(feat_hbm.at[idx_at(b)], rows_v.at[b], sems[b])
        for b in range(NBUF)
    ]

    def ring_step(it, _):
        i0 = it * NBUF
        for b in range(NBUF):
            i = i0 + b
            waiters[b].wait()
            for d8 in range(D // LANES):
                sl = pl.ds(d8 * LANES, LANES)
                a = rows_v[b, 0, sl]
                for k in range(1, K):
                    a = a + rows_v[b, k, sl]
                acc_v[i, sl] = a
            nxt = i + NBUF

            @pl.when(nxt < P)
            def _fire():
                pltpu.async_copy(
                    feat_hbm.at[idx_at(nxt)], rows_v.at[b], sems[b])
        return _

    lax.fori_loop(0, P // NBUF, ring_step, None)

    pltpu.sync_copy(acc_v, acc_hbm.at[pl.ds(base, P)])
    pltpu.sync_copy(cnt_v, cnt_hbm.at[pl.ds(base, P)])


def _sc_gather(features, nbrs_pad, rowsum):
    mesh = plsc.VectorSubcoreMesh(core_axis_name="c", subcore_axis_name="s")
    run = pl.kernel(
        _sc_body,
        out_type=(
            jax.ShapeDtypeStruct((NPAD, D), jnp.float32),
            jax.ShapeDtypeStruct((NPAD,), jnp.float32),
        ),
        mesh=mesh,
        scratch_types=[
            pltpu.VMEM((P * K,), jnp.int32),
            pltpu.VMEM((N,), jnp.float32),
            pltpu.VMEM((NBUF, K, D), jnp.float32),
            pltpu.VMEM((P, D), jnp.float32),
            pltpu.VMEM((P,), jnp.float32),
        ] + [pltpu.SemaphoreType.DMA] * NBUF,
        compiler_params=pltpu.CompilerParams(needs_layout_passes=False),
    )
    return run(features, nbrs_pad, rowsum)


def _finish_body(f_ref, a_ref, c_ref, m_ref, o_ref):
    m = m_ref[0, 0] + 1e-6
    f = f_ref[:] / m
    num = jnp.maximum(c_ref[:], 1.0)
    mean = a_ref[:] / m / num
    x = f - mean
    local = jnp.maximum(x, 0.0) + jnp.log1p(jnp.exp(-jnp.abs(x)))
    dmax = jnp.max(f, axis=1, keepdims=True)
    o_ref[:] = jnp.max(local * (f / (1e-6 + dmax)), axis=1, keepdims=True)


def _finish(features, acc, cnt, mx):
    return pl.pallas_call(
        _finish_body,
        out_shape=jax.ShapeDtypeStruct((N, 1), jnp.float32),
    )(features, acc, cnt, mx)


@jax.jit
def _run(features, neighbors):
    rowsum2d, mx = _row_stats(features)
    rowsum = rowsum2d.reshape(N)
    nbrs_pad = jnp.pad(neighbors, ((0, NPAD - N), (0, 0))).reshape(NPAD * K)
    acc, cnt = _sc_gather(features, nbrs_pad, rowsum)
    return _finish(features, acc[:N], cnt[:N].reshape(N, 1), mx)


def kernel(features, neighbors, first_pcd_length, second_pcd_length):
    return _run(features, neighbors)
